# Initial kernel scaffold; baseline (speedup 1.0000x reference)
import jax
import jax.numpy as jnp
from jax.experimental import pallas as pl


def kernel(x, W0, b0, g0, be0, W1, b1, g1, be1, Wp, bp):
    raise NotImplementedError("write your pallas kernel here")


# speedup vs baseline: 5.0468x; 5.0468x over previous
import jax
import jax.numpy as jnp
import numpy as np
from jax.experimental import pallas as pl
from jax.experimental.pallas import tpu as pltpu

_N_BINS = 8
_BOUND = 3.0
_MIN_BIN_W = 1e-3
_MIN_BIN_H = 1e-3
_MIN_DERIV = 1e-3
_LN_EPS = 1e-6
_R = 512

_HI = jax.lax.Precision.HIGHEST


def _softplus(x):
    return jnp.maximum(x, 0.0) + jnp.log1p(jnp.exp(-jnp.abs(x)))


def _softmax8(es):
    m = es[0]
    for e in es[1:]:
        m = jnp.maximum(m, e)
    ex = [jnp.exp(e - m) for e in es]
    s = ex[0]
    for e in ex[1:]:
        s = s + e
    r = 1.0 / s
    return [e * r for e in ex]


def _ln_relu(pre, g, be):
    hid = pre.shape[0]
    ones = jnp.full((1, hid), 1.0 / hid, jnp.float32)
    mu = jnp.dot(ones, pre, preferred_element_type=jnp.float32)
    cent = pre - mu
    var = jnp.dot(ones, cent * cent, preferred_element_type=jnp.float32)
    normed = cent * jax.lax.rsqrt(var + _LN_EPS) * g + be
    return jnp.maximum(normed, 0.0)


def _body(x_ref, dt_ref, w0t_ref, b0_ref, g0_ref, be0_ref,
          w1t_ref, b1_ref, g1_ref, be1_ref, wpt_ref, bp_ref,
          y_ref, ld_ref):
    K = _N_BINS
    x_blk = x_ref[...]
    dt = dt_ref[...]
    xt = jax.lax.dot_general(dt, x_blk, (((1,), (1,)), ((), ())),
                             precision=_HI, preferred_element_type=jnp.float32)
    half = x_blk.shape[1] // 2
    xm = xt[0:half, :]
    xu = xt[half:2 * half, :]

    pre0 = jnp.dot(w0t_ref[...], xm, preferred_element_type=jnp.float32) + b0_ref[...]
    h1 = _ln_relu(pre0, g0_ref[...], be0_ref[...])
    pre1 = jnp.dot(w1t_ref[...], h1, preferred_element_type=jnp.float32) + b1_ref[...]
    h2 = _ln_relu(pre1, g1_ref[...], be1_ref[...]) + h1
    raw = jnp.dot(wpt_ref[...], h2, preferred_element_type=jnp.float32) + bp_ref[...]

    wr = [raw[half * k:half * (k + 1), :] for k in range(K)]
    hr = [raw[half * (K + k):half * (K + k + 1), :] for k in range(K)]
    dr = [raw[half * (2 * K + k):half * (2 * K + k + 1), :] for k in range(K + 1)]

    cw1 = (1.0 - _MIN_BIN_W * K) * (2.0 * _BOUND)
    cw0 = _MIN_BIN_W * (2.0 * _BOUND)
    wk = [p * cw1 + cw0 for p in _softmax8(wr)]
    ch1 = (1.0 - _MIN_BIN_H * K) * (2.0 * _BOUND)
    ch0 = _MIN_BIN_H * (2.0 * _BOUND)
    hk = [p * ch1 + ch0 for p in _softmax8(hr)]
    dk = [_softplus(d) + _MIN_DERIV for d in dr]

    cw = [None] * (K + 1)
    ch = [None] * (K + 1)
    acc_w = wk[0] - _BOUND
    acc_h = hk[0] - _BOUND
    for k in range(1, K):
        cw[k] = acc_w
        ch[k] = acc_h
        acc_w = acc_w + wk[k]
        acc_h = acc_h + hk[k]

    xc = jnp.clip(xu, -_BOUND, _BOUND)
    sel_w, sel_h = wk[0], hk[0]
    sel_x = jnp.full_like(xc, -_BOUND)
    sel_y = jnp.full_like(xc, -_BOUND)
    sel_d0, sel_d1 = dk[0], dk[1]
    for k in range(1, K):
        m = xc >= cw[k]
        sel_w = jnp.where(m, wk[k], sel_w)
        sel_h = jnp.where(m, hk[k], sel_h)
        sel_x = jnp.where(m, cw[k], sel_x)
        sel_y = jnp.where(m, ch[k], sel_y)
        sel_d0 = jnp.where(m, dk[k], sel_d0)
        sel_d1 = jnp.where(m, dk[k + 1], sel_d1)

    rw = 1.0 / sel_w
    s = sel_h * rw
    theta = (xc - sel_x) * rw
    omt = 1.0 - theta
    t1m = theta * omt
    th2 = theta * theta
    den = s + (sel_d1 + sel_d0 - 2.0 * s) * t1m
    y_in = sel_y + sel_h * (s * th2 + sel_d0 * t1m) / den
    ld_in = (2.0 * jnp.log(s)
             + jnp.log(sel_d1 * th2 + 2.0 * s * t1m + sel_d0 * omt * omt)
             - 2.0 * jnp.log(den))
    inside = (xu > -_BOUND) & (xu < _BOUND)
    yu = jnp.where(inside, y_in, xu)
    ld = jnp.where(inside, ld_in, 0.0)

    ones = jnp.full((1, half), 1.0, jnp.float32)
    ld_ref[...] = jnp.dot(ones, ld, preferred_element_type=jnp.float32)

    yt = jnp.concatenate([xm, yu], axis=0)
    y_ref[...] = jax.lax.dot_general(yt, dt, (((0,), (0,)), ((), ())),
                                     precision=_HI,
                                     preferred_element_type=jnp.float32)


def _run(x, dt, w0t, b0c, g0c, be0c, w1t, b1c, g1c, be1c, wpt, bpc,
         interpret=False):
    B, D = x.shape
    hid = w0t.shape[0]
    npar = wpt.shape[0]
    grid = (B // _R,)
    full = lambda shape: pl.BlockSpec(shape, lambda i: (0, 0))
    y, ld = pl.pallas_call(
        _body,
        grid=grid,
        in_specs=[
            pl.BlockSpec((_R, D), lambda i: (i, 0)),
            full((D, D)),
            full((hid, D // 2)), full((hid, 1)), full((hid, 1)), full((hid, 1)),
            full((hid, hid)), full((hid, 1)), full((hid, 1)), full((hid, 1)),
            full((npar, hid)), full((npar, 1)),
        ],
        out_specs=[
            pl.BlockSpec((_R, D), lambda i: (i, 0)),
            pl.BlockSpec((1, _R), lambda i: (0, i)),
        ],
        out_shape=[
            jax.ShapeDtypeStruct((B, D), jnp.float32),
            jax.ShapeDtypeStruct((1, B), jnp.float32),
        ],
        compiler_params=pltpu.CompilerParams(
            dimension_semantics=("parallel",)),
        interpret=interpret,
    )(x, dt, w0t, b0c, g0c, be0c, w1t, b1c, g1c, be1c, wpt, bpc)
    return y, ld[0]


def kernel(x, W0, b0, g0, be0, W1, b1, g1, be1, Wp, bp):
    B, D = x.shape
    half = D // 2
    hid = W0.shape[1]
    npp = 3 * _N_BINS + 1

    dt_np = np.zeros((D, D), np.float32)
    for u in range(half):
        dt_np[u, 2 * u] = 1.0
        dt_np[half + u, 2 * u + 1] = 1.0
    dt = jnp.asarray(dt_np)

    w0t = W0.T
    w1t = W1.T
    wpt = Wp.reshape(hid, half, npp).transpose(2, 1, 0).reshape(npp * half, hid)
    bpc = bp.reshape(half, npp).T.reshape(npp * half, 1)

    return _run(x, dt, w0t, b0[:, None], g0[:, None], be0[:, None],
                w1t, b1[:, None], g1[:, None], be1[:, None], wpt, bpc)


# speedup vs baseline: 8.0458x; 1.5942x over previous
import jax
import jax.numpy as jnp
import numpy as np
from jax.experimental import pallas as pl
from jax.experimental.pallas import tpu as pltpu

_N_BINS = 8
_BOUND = 3.0
_MIN_BIN_W = 1e-3
_MIN_BIN_H = 1e-3
_MIN_DERIV = 1e-3
_LN_EPS = 1e-6
_R = 1024


def _softplus(x):
    return jnp.maximum(x, 0.0) + jnp.log1p(jnp.exp(-jnp.abs(x)))


def _softmax8(es):
    m = es[0]
    for e in es[1:]:
        m = jnp.maximum(m, e)
    ex = [jnp.exp(e - m) for e in es]
    s = ex[0]
    for e in ex[1:]:
        s = s + e
    r = 1.0 / s
    return [e * r for e in ex]


def _ln_relu(pre, g, be):
    hid = pre.shape[0]
    ones = jnp.full((1, hid), 1.0 / hid, jnp.float32)
    mu = jnp.dot(ones, pre, preferred_element_type=jnp.float32)
    cent = pre - mu
    var = jnp.dot(ones, cent * cent, preferred_element_type=jnp.float32)
    normed = cent * jax.lax.rsqrt(var + _LN_EPS) * g + be
    return jnp.maximum(normed, 0.0)


def _body(x_ref, dt_ref, w0t_ref, b0_ref, g0_ref, be0_ref,
          w1t_ref, b1_ref, g1_ref, be1_ref, wpt_ref, bp_ref,
          y_ref, ld_ref):
    K = _N_BINS
    x_blk = x_ref[...]
    dt = dt_ref[...]
    xt = jax.lax.dot_general(dt, x_blk, (((1,), (1,)), ((), ())),
                             preferred_element_type=jnp.float32)
    half = x_blk.shape[1] // 2
    xm = xt[0:half, :]
    xu = xt[half:2 * half, :]

    pre0 = jnp.dot(w0t_ref[...], xm, preferred_element_type=jnp.float32) + b0_ref[...]
    h1 = _ln_relu(pre0, g0_ref[...], be0_ref[...])
    pre1 = jnp.dot(w1t_ref[...], h1, preferred_element_type=jnp.float32) + b1_ref[...]
    h2 = _ln_relu(pre1, g1_ref[...], be1_ref[...]) + h1
    raw = jnp.dot(wpt_ref[...], h2, preferred_element_type=jnp.float32) + bp_ref[...]

    wr = [raw[half * k:half * (k + 1), :] for k in range(K)]
    hr = [raw[half * (K + k):half * (K + k + 1), :] for k in range(K)]
    dr = [raw[half * (2 * K + k):half * (2 * K + k + 1), :] for k in range(K + 1)]

    cw1 = (1.0 - _MIN_BIN_W * K) * (2.0 * _BOUND)
    cw0 = _MIN_BIN_W * (2.0 * _BOUND)
    wk = [p * cw1 + cw0 for p in _softmax8(wr)]
    ch1 = (1.0 - _MIN_BIN_H * K) * (2.0 * _BOUND)
    ch0 = _MIN_BIN_H * (2.0 * _BOUND)
    hk = [p * ch1 + ch0 for p in _softmax8(hr)]
    dk = [_softplus(d) + _MIN_DERIV for d in dr]

    cw = [None] * (K + 1)
    ch = [None] * (K + 1)
    acc_w = wk[0] - _BOUND
    acc_h = hk[0] - _BOUND
    for k in range(1, K):
        cw[k] = acc_w
        ch[k] = acc_h
        acc_w = acc_w + wk[k]
        acc_h = acc_h + hk[k]

    xc = jnp.clip(xu, -_BOUND, _BOUND)
    sel_w, sel_h = wk[0], hk[0]
    sel_x = jnp.full_like(xc, -_BOUND)
    sel_y = jnp.full_like(xc, -_BOUND)
    sel_d0, sel_d1 = dk[0], dk[1]
    for k in range(1, K):
        m = xc >= cw[k]
        sel_w = jnp.where(m, wk[k], sel_w)
        sel_h = jnp.where(m, hk[k], sel_h)
        sel_x = jnp.where(m, cw[k], sel_x)
        sel_y = jnp.where(m, ch[k], sel_y)
        sel_d0 = jnp.where(m, dk[k], sel_d0)
        sel_d1 = jnp.where(m, dk[k + 1], sel_d1)

    rw = 1.0 / sel_w
    s = sel_h * rw
    theta = (xc - sel_x) * rw
    omt = 1.0 - theta
    t1m = theta * omt
    th2 = theta * theta
    den = s + (sel_d1 + sel_d0 - 2.0 * s) * t1m
    y_in = sel_y + sel_h * (s * th2 + sel_d0 * t1m) / den
    ld_in = (2.0 * jnp.log(s)
             + jnp.log(sel_d1 * th2 + 2.0 * s * t1m + sel_d0 * omt * omt)
             - 2.0 * jnp.log(den))
    inside = (xu > -_BOUND) & (xu < _BOUND)
    yu = jnp.where(inside, y_in, xu)
    ld = jnp.where(inside, ld_in, 0.0)

    ones = jnp.full((1, half), 1.0, jnp.float32)
    ld_ref[...] = jnp.dot(ones, ld, preferred_element_type=jnp.float32)

    yt = jnp.concatenate([xm, yu], axis=0)
    y_ref[...] = jax.lax.dot_general(yt, dt, (((0,), (0,)), ((), ())),
                                     preferred_element_type=jnp.float32)


def _run(x, dt, w0t, b0c, g0c, be0c, w1t, b1c, g1c, be1c, wpt, bpc,
         interpret=False):
    B, D = x.shape
    hid = w0t.shape[0]
    npar = wpt.shape[0]
    grid = (B // _R,)
    full = lambda shape: pl.BlockSpec(shape, lambda i: (0, 0))
    y, ld = pl.pallas_call(
        _body,
        grid=grid,
        in_specs=[
            pl.BlockSpec((_R, D), lambda i: (i, 0)),
            full((D, D)),
            full((hid, D // 2)), full((hid, 1)), full((hid, 1)), full((hid, 1)),
            full((hid, hid)), full((hid, 1)), full((hid, 1)), full((hid, 1)),
            full((npar, hid)), full((npar, 1)),
        ],
        out_specs=[
            pl.BlockSpec((_R, D), lambda i: (i, 0)),
            pl.BlockSpec((1, _R), lambda i: (0, i)),
        ],
        out_shape=[
            jax.ShapeDtypeStruct((B, D), jnp.float32),
            jax.ShapeDtypeStruct((1, B), jnp.float32),
        ],
        compiler_params=pltpu.CompilerParams(
            dimension_semantics=("parallel",)),
        interpret=interpret,
    )(x, dt, w0t, b0c, g0c, be0c, w1t, b1c, g1c, be1c, wpt, bpc)
    return y, ld[0]


def kernel(x, W0, b0, g0, be0, W1, b1, g1, be1, Wp, bp):
    B, D = x.shape
    half = D // 2
    hid = W0.shape[1]
    npp = 3 * _N_BINS + 1

    dt_np = np.zeros((D, D), np.float32)
    for u in range(half):
        dt_np[u, 2 * u] = 1.0
        dt_np[half + u, 2 * u + 1] = 1.0
    dt = jnp.asarray(dt_np)

    w0t = W0.T
    w1t = W1.T
    wpt = Wp.reshape(hid, half, npp).transpose(2, 1, 0).reshape(npp * half, hid)
    bpc = bp.reshape(half, npp).T.reshape(npp * half, 1)

    return _run(x, dt, w0t, b0[:, None], g0[:, None], be0[:, None],
                w1t, b1[:, None], g1[:, None], be1[:, None], wpt, bpc)


# speedup vs baseline: 12.1042x; 1.5044x over previous
import jax
import jax.numpy as jnp
import numpy as np
from jax.experimental import pallas as pl
from jax.experimental.pallas import tpu as pltpu

_N_BINS = 8
_BOUND = 3.0
_MIN_BIN_W = 1e-3
_MIN_BIN_H = 1e-3
_MIN_DERIV = 1e-3
_LN_EPS = 1e-6
_R = 2048


def _softplus(x):
    return jnp.maximum(x, 0.0) + jnp.log(1.0 + jnp.exp(-jnp.abs(x)))


def _softmax8_affine(es, scale, offset):
    ex = [jnp.exp(e) for e in es]
    s = ex[0]
    for e in ex[1:]:
        s = s + e
    r = scale / s
    return [e * r + offset for e in ex]


def _ln_relu(pre, g, be):
    hid = pre.shape[0]
    ones = jnp.full((1, hid), 1.0 / hid, jnp.float32)
    mu = jnp.dot(ones, pre, preferred_element_type=jnp.float32)
    cent = pre - mu
    var = jnp.dot(ones, cent * cent, preferred_element_type=jnp.float32)
    normed = cent * jax.lax.rsqrt(var + _LN_EPS) * g + be
    return jnp.maximum(normed, 0.0)


def _body(x_ref, dt_ref, w0t_ref, g0_ref, be0_ref,
          w1t_ref, g1_ref, be1_ref, wpt_ref,
          y_ref, ld_ref):
    K = _N_BINS
    x_blk = x_ref[...]
    dt = dt_ref[...]
    xt = jax.lax.dot_general(dt, x_blk, (((1,), (1,)), ((), ())),
                             preferred_element_type=jnp.float32)
    half = x_blk.shape[1] // 2
    xm = xt[0:half, :]
    xu = xt[half:2 * half, :]
    ones_row = jnp.full((1, x_blk.shape[0]), 1.0, jnp.float32)

    pre0 = jnp.dot(w0t_ref[...], jnp.concatenate([xm, ones_row], axis=0),
                   preferred_element_type=jnp.float32)
    h1 = _ln_relu(pre0, g0_ref[...], be0_ref[...])
    pre1 = jnp.dot(w1t_ref[...], jnp.concatenate([h1, ones_row], axis=0),
                   preferred_element_type=jnp.float32)
    h2 = _ln_relu(pre1, g1_ref[...], be1_ref[...]) + h1
    raw = jnp.dot(wpt_ref[...], jnp.concatenate([h2, ones_row], axis=0),
                  preferred_element_type=jnp.float32)

    wr = [raw[half * k:half * (k + 1), :] for k in range(K)]
    hr = [raw[half * (K + k):half * (K + k + 1), :] for k in range(K)]
    dr = [raw[half * (2 * K + k):half * (2 * K + k + 1), :] for k in range(K + 1)]

    wk = _softmax8_affine(wr, (1.0 - _MIN_BIN_W * K) * (2.0 * _BOUND),
                          _MIN_BIN_W * (2.0 * _BOUND))
    hk = _softmax8_affine(hr, (1.0 - _MIN_BIN_H * K) * (2.0 * _BOUND),
                          _MIN_BIN_H * (2.0 * _BOUND))

    cw = [None] * (K + 1)
    ch = [None] * (K + 1)
    acc_w = wk[0] - _BOUND
    acc_h = hk[0] - _BOUND
    for k in range(1, K):
        cw[k] = acc_w
        ch[k] = acc_h
        acc_w = acc_w + wk[k]
        acc_h = acc_h + hk[k]

    xc = jax.lax.clamp(-_BOUND, xu, _BOUND)
    sel_w, sel_h = wk[0], hk[0]
    sel_x = jnp.full_like(xc, -_BOUND)
    sel_y = jnp.full_like(xc, -_BOUND)
    sel_rd0, sel_rd1 = dr[0], dr[1]
    for k in range(1, K):
        m = xc >= cw[k]
        sel_w = jnp.where(m, wk[k], sel_w)
        sel_h = jnp.where(m, hk[k], sel_h)
        sel_x = jnp.where(m, cw[k], sel_x)
        sel_y = jnp.where(m, ch[k], sel_y)
        sel_rd0 = jnp.where(m, dr[k], sel_rd0)
        sel_rd1 = jnp.where(m, dr[k + 1], sel_rd1)
    sel_d0 = _softplus(sel_rd0) + _MIN_DERIV
    sel_d1 = _softplus(sel_rd1) + _MIN_DERIV

    rw = 1.0 / sel_w
    s = sel_h * rw
    theta = (xc - sel_x) * rw
    omt = 1.0 - theta
    t1m = theta * omt
    th2 = theta * theta
    den = s + (sel_d1 + sel_d0 - 2.0 * s) * t1m
    rden = 1.0 / den
    y_in = sel_y + sel_h * (s * th2 + sel_d0 * t1m) * rden
    numer = sel_d1 * th2 + 2.0 * s * t1m + sel_d0 * omt * omt
    ld_in = jnp.log(s * s * numer * (rden * rden))
    inside = (xu > -_BOUND) & (xu < _BOUND)
    yu = jnp.where(inside, y_in, xu)
    ld = jnp.where(inside, ld_in, 0.0)

    ld_ref[...] = jnp.dot(jnp.full((1, half), 1.0, jnp.float32), ld,
                          preferred_element_type=jnp.float32)

    yt = jnp.concatenate([xm, yu], axis=0)
    y_ref[...] = jax.lax.dot_general(yt, dt, (((0,), (0,)), ((), ())),
                                     preferred_element_type=jnp.float32)


def _run(x, dt, w0t, g0c, be0c, w1t, g1c, be1c, wpt, interpret=False):
    B, D = x.shape
    hid = w0t.shape[0]
    npar = wpt.shape[0]
    grid = (B // _R,)
    full = lambda shape: pl.BlockSpec(shape, lambda i: (0, 0))
    y, ld = pl.pallas_call(
        _body,
        grid=grid,
        in_specs=[
            pl.BlockSpec((_R, D), lambda i: (i, 0)),
            full((D, D)),
            full((hid, D // 2 + 1)), full((hid, 1)), full((hid, 1)),
            full((hid, hid + 1)), full((hid, 1)), full((hid, 1)),
            full((npar, hid + 1)),
        ],
        out_specs=[
            pl.BlockSpec((_R, D), lambda i: (i, 0)),
            pl.BlockSpec((1, _R), lambda i: (0, i)),
        ],
        out_shape=[
            jax.ShapeDtypeStruct((B, D), jnp.float32),
            jax.ShapeDtypeStruct((1, B), jnp.float32),
        ],
        compiler_params=pltpu.CompilerParams(
            dimension_semantics=("parallel",)),
        interpret=interpret,
    )(x, dt, w0t, g0c, be0c, w1t, g1c, be1c, wpt)
    return y, ld[0]


def kernel(x, W0, b0, g0, be0, W1, b1, g1, be1, Wp, bp):
    B, D = x.shape
    half = D // 2
    hid = W0.shape[1]
    npp = 3 * _N_BINS + 1

    dt_np = np.zeros((D, D), np.float32)
    for u in range(half):
        dt_np[u, 2 * u] = 1.0
        dt_np[half + u, 2 * u + 1] = 1.0
    dt = jnp.asarray(dt_np)

    w0t = jnp.concatenate([W0.T, b0[:, None]], axis=1)
    w1t = jnp.concatenate([W1.T, b1[:, None]], axis=1)
    wpt = Wp.reshape(hid, half, npp).transpose(2, 1, 0).reshape(npp * half, hid)
    bpc = bp.reshape(half, npp).T.reshape(npp * half, 1)
    wpt = jnp.concatenate([wpt, bpc], axis=1)

    return _run(x, dt, w0t, g0[:, None], be0[:, None],
                w1t, g1[:, None], be1[:, None], wpt)


# speedup vs baseline: 15.7354x; 1.3000x over previous
import jax
import jax.numpy as jnp
import numpy as np
from jax.experimental import pallas as pl
from jax.experimental.pallas import tpu as pltpu

_N_BINS = 8
_BOUND = 3.0
_MIN_BIN_W = 1e-3
_MIN_BIN_H = 1e-3
_MIN_DERIV = 1e-3
_LN_EPS = 1e-6
_R = 2048


def _softplus(x):
    return jnp.maximum(x, 0.0) + jnp.log(1.0 + jnp.exp(-jnp.abs(x)))


def _softmax8_affine(es, scale, offset):
    ex = [jnp.exp(e) for e in es]
    s = ex[0]
    for e in ex[1:]:
        s = s + e
    r = scale / s
    return [e * r + offset for e in ex]


def _ln_relu(pre, g, be):
    hid = pre.shape[0]
    ones = jnp.full((1, hid), 1.0 / hid, jnp.float32)
    mu = jnp.dot(ones, pre, preferred_element_type=jnp.float32)
    cent = pre - mu
    var = jnp.dot(ones, cent * cent, preferred_element_type=jnp.float32)
    normed = cent * jax.lax.rsqrt(var + _LN_EPS) * g + be
    return jnp.maximum(normed, 0.0)


def _body(x_ref, dt_ref, w0t_ref, g0_ref, be0_ref,
          w1t_ref, g1_ref, be1_ref, wpt_ref,
          y_ref, ld_ref):
    K = _N_BINS
    x_blk = x_ref[...]
    dt = dt_ref[...]
    xt = jnp.dot(dt, x_blk, preferred_element_type=jnp.float32)
    half = x_blk.shape[0] // 2
    xm = xt[0:half, :]
    xu = xt[half:2 * half, :]
    ones_row = jnp.full((1, x_blk.shape[1]), 1.0, jnp.float32)

    pre0 = jnp.dot(w0t_ref[...], jnp.concatenate([xm, ones_row], axis=0),
                   preferred_element_type=jnp.float32)
    h1 = _ln_relu(pre0, g0_ref[...], be0_ref[...])
    pre1 = jnp.dot(w1t_ref[...], jnp.concatenate([h1, ones_row], axis=0),
                   preferred_element_type=jnp.float32)
    h2 = _ln_relu(pre1, g1_ref[...], be1_ref[...]) + h1
    raw = jnp.dot(wpt_ref[...], jnp.concatenate([h2, ones_row], axis=0),
                  preferred_element_type=jnp.float32)

    wr = [raw[half * k:half * (k + 1), :] for k in range(K)]
    hr = [raw[half * (K + k):half * (K + k + 1), :] for k in range(K)]
    dr = [raw[half * (2 * K + k):half * (2 * K + k + 1), :] for k in range(K + 1)]

    wk = _softmax8_affine(wr, (1.0 - _MIN_BIN_W * K) * (2.0 * _BOUND),
                          _MIN_BIN_W * (2.0 * _BOUND))
    hk = _softmax8_affine(hr, (1.0 - _MIN_BIN_H * K) * (2.0 * _BOUND),
                          _MIN_BIN_H * (2.0 * _BOUND))

    cw = [None] * (K + 1)
    ch = [None] * (K + 1)
    acc_w = wk[0] - _BOUND
    acc_h = hk[0] - _BOUND
    for k in range(1, K):
        cw[k] = acc_w
        ch[k] = acc_h
        acc_w = acc_w + wk[k]
        acc_h = acc_h + hk[k]

    xc = jax.lax.clamp(-_BOUND, xu, _BOUND)
    sel_w, sel_h = wk[0], hk[0]
    sel_x = jnp.full_like(xc, -_BOUND)
    sel_y = jnp.full_like(xc, -_BOUND)
    sel_rd0, sel_rd1 = dr[0], dr[1]
    for k in range(1, K):
        m = xc >= cw[k]
        sel_w = jnp.where(m, wk[k], sel_w)
        sel_h = jnp.where(m, hk[k], sel_h)
        sel_x = jnp.where(m, cw[k], sel_x)
        sel_y = jnp.where(m, ch[k], sel_y)
        sel_rd0 = jnp.where(m, dr[k], sel_rd0)
        sel_rd1 = jnp.where(m, dr[k + 1], sel_rd1)
    sel_d0 = _softplus(sel_rd0) + _MIN_DERIV
    sel_d1 = _softplus(sel_rd1) + _MIN_DERIV

    rw = 1.0 / sel_w
    s = sel_h * rw
    theta = (xc - sel_x) * rw
    omt = 1.0 - theta
    t1m = theta * omt
    th2 = theta * theta
    den = s + (sel_d1 + sel_d0 - 2.0 * s) * t1m
    rden = 1.0 / den
    y_in = sel_y + sel_h * (s * th2 + sel_d0 * t1m) * rden
    numer = sel_d1 * th2 + 2.0 * s * t1m + sel_d0 * omt * omt
    ld_in = jnp.log(s * s * numer * (rden * rden))
    inside = (xu > -_BOUND) & (xu < _BOUND)
    yu = jnp.where(inside, y_in, xu)
    ld = jnp.where(inside, ld_in, 0.0)

    ld_ref[...] = jnp.dot(jnp.full((1, half), 1.0, jnp.float32), ld,
                          preferred_element_type=jnp.float32)

    yt = jnp.concatenate([xm, yu], axis=0)
    y_ref[...] = jax.lax.dot_general(dt, yt, (((0,), (0,)), ((), ())),
                                     preferred_element_type=jnp.float32)


def _run(xT, dt, w0t, g0c, be0c, w1t, g1c, be1c, wpt, interpret=False):
    D, B = xT.shape
    hid = w0t.shape[0]
    npar = wpt.shape[0]
    grid = (B // _R,)
    full = lambda shape: pl.BlockSpec(shape, lambda i: (0, 0))
    yT, ld = pl.pallas_call(
        _body,
        grid=grid,
        in_specs=[
            pl.BlockSpec((D, _R), lambda i: (0, i)),
            full((D, D)),
            full((hid, D // 2 + 1)), full((hid, 1)), full((hid, 1)),
            full((hid, hid + 1)), full((hid, 1)), full((hid, 1)),
            full((npar, hid + 1)),
        ],
        out_specs=[
            pl.BlockSpec((D, _R), lambda i: (0, i)),
            pl.BlockSpec((1, _R), lambda i: (0, i)),
        ],
        out_shape=[
            jax.ShapeDtypeStruct((D, B), jnp.float32),
            jax.ShapeDtypeStruct((1, B), jnp.float32),
        ],
        compiler_params=pltpu.CompilerParams(
            dimension_semantics=("parallel",)),
        interpret=interpret,
    )(xT, dt, w0t, g0c, be0c, w1t, g1c, be1c, wpt)
    return yT, ld[0]


def kernel(x, W0, b0, g0, be0, W1, b1, g1, be1, Wp, bp):
    B, D = x.shape
    half = D // 2
    hid = W0.shape[1]
    npp = 3 * _N_BINS + 1

    dt_np = np.zeros((D, D), np.float32)
    for u in range(half):
        dt_np[u, 2 * u] = 1.0
        dt_np[half + u, 2 * u + 1] = 1.0
    dt = jnp.asarray(dt_np)

    w0t = jnp.concatenate([W0.T, b0[:, None]], axis=1)
    w1t = jnp.concatenate([W1.T, b1[:, None]], axis=1)
    wpt = Wp.reshape(hid, half, npp).transpose(2, 1, 0).reshape(npp * half, hid)
    bpc = bp.reshape(half, npp).T.reshape(npp * half, 1)
    wpt = jnp.concatenate([wpt, bpc], axis=1)

    yT, ld = _run(x.T, dt, w0t, g0[:, None], be0[:, None],
                  w1t, g1[:, None], be1[:, None], wpt)
    return yT.T, ld


# speedup vs baseline: 17.1274x; 1.0885x over previous
import jax
import jax.numpy as jnp
import numpy as np
from jax.experimental import pallas as pl
from jax.experimental.pallas import tpu as pltpu

_N_BINS = 8
_BOUND = 3.0
_MIN_BIN_W = 1e-3
_MIN_BIN_H = 1e-3
_MIN_DERIV = 1e-3
_LN_EPS = 1e-6
_R = 4096


def _softplus(x):
    return jnp.maximum(x, 0.0) + jnp.log(1.0 + jnp.exp(-jnp.abs(x)))


def _softmax8_affine(es, scale, offset):
    ex = [jnp.exp(e) for e in es]
    s = ex[0]
    for e in ex[1:]:
        s = s + e
    r = scale / s
    return [e * r + offset for e in ex]


def _ln_relu(pre, g, be):
    hid = pre.shape[0]
    ones = jnp.full((1, hid), 1.0 / hid, jnp.float32)
    mu = jnp.dot(ones, pre, preferred_element_type=jnp.float32)
    cent = pre - mu
    var = jnp.dot(ones, cent * cent, preferred_element_type=jnp.float32)
    normed = cent * jax.lax.rsqrt(var + _LN_EPS) * g + be
    return jnp.maximum(normed, 0.0)


def _body(x_ref, dt_ref, w0t_ref, g0_ref, be0_ref,
          w1t_ref, g1_ref, be1_ref, wpt_ref,
          y_ref, ld_ref):
    K = _N_BINS
    x_blk = x_ref[...]
    dt = dt_ref[...]
    xt = jnp.dot(dt, x_blk, preferred_element_type=jnp.float32)
    half = x_blk.shape[0] // 2
    xm = xt[0:half, :]
    xu = xt[half:2 * half, :]
    ones_row = jnp.full((1, x_blk.shape[1]), 1.0, jnp.float32)

    pre0 = jnp.dot(w0t_ref[...], jnp.concatenate([xm, ones_row], axis=0),
                   preferred_element_type=jnp.float32)
    h1 = _ln_relu(pre0, g0_ref[...], be0_ref[...])
    pre1 = jnp.dot(w1t_ref[...], jnp.concatenate([h1, ones_row], axis=0),
                   preferred_element_type=jnp.float32)
    h2 = _ln_relu(pre1, g1_ref[...], be1_ref[...]) + h1
    raw = jnp.dot(wpt_ref[...], jnp.concatenate([h2, ones_row], axis=0),
                  preferred_element_type=jnp.float32)

    wr = [raw[half * k:half * (k + 1), :] for k in range(K)]
    hr = [raw[half * (K + k):half * (K + k + 1), :] for k in range(K)]
    dr = [raw[half * (2 * K + k):half * (2 * K + k + 1), :] for k in range(K + 1)]

    wk = _softmax8_affine(wr, (1.0 - _MIN_BIN_W * K) * (2.0 * _BOUND),
                          _MIN_BIN_W * (2.0 * _BOUND))
    hk = _softmax8_affine(hr, (1.0 - _MIN_BIN_H * K) * (2.0 * _BOUND),
                          _MIN_BIN_H * (2.0 * _BOUND))

    cw = [None] * (K + 1)
    ch = [None] * (K + 1)
    acc_w = wk[0] - _BOUND
    acc_h = hk[0] - _BOUND
    for k in range(1, K):
        cw[k] = acc_w
        ch[k] = acc_h
        acc_w = acc_w + wk[k]
        acc_h = acc_h + hk[k]

    xc = jax.lax.clamp(-_BOUND, xu, _BOUND)
    sel_w, sel_h = wk[0], hk[0]
    sel_x = jnp.full_like(xc, -_BOUND)
    sel_y = jnp.full_like(xc, -_BOUND)
    sel_rd0, sel_rd1 = dr[0], dr[1]
    for k in range(1, K):
        m = xc >= cw[k]
        sel_w = jnp.where(m, wk[k], sel_w)
        sel_h = jnp.where(m, hk[k], sel_h)
        sel_x = jnp.where(m, cw[k], sel_x)
        sel_y = jnp.where(m, ch[k], sel_y)
        sel_rd0 = jnp.where(m, dr[k], sel_rd0)
        sel_rd1 = jnp.where(m, dr[k + 1], sel_rd1)
    sel_d0 = _softplus(sel_rd0) + _MIN_DERIV
    sel_d1 = _softplus(sel_rd1) + _MIN_DERIV

    rw = 1.0 / sel_w
    s = sel_h * rw
    theta = (xc - sel_x) * rw
    omt = 1.0 - theta
    t1m = theta * omt
    th2 = theta * theta
    den = s + (sel_d1 + sel_d0 - 2.0 * s) * t1m
    rden = 1.0 / den
    y_in = sel_y + sel_h * (s * th2 + sel_d0 * t1m) * rden
    numer = sel_d1 * th2 + 2.0 * s * t1m + sel_d0 * omt * omt
    ld_in = jnp.log(s * s * numer * (rden * rden))
    inside = (xu > -_BOUND) & (xu < _BOUND)
    yu = jnp.where(inside, y_in, xu)
    ld = jnp.where(inside, ld_in, 0.0)

    ld_ref[...] = jnp.dot(jnp.full((1, half), 1.0, jnp.float32), ld,
                          preferred_element_type=jnp.float32)

    yt = jnp.concatenate([xm, yu], axis=0)
    y_ref[...] = jax.lax.dot_general(dt, yt, (((0,), (0,)), ((), ())),
                                     preferred_element_type=jnp.float32)


def _run(xT, dt, w0t, g0c, be0c, w1t, g1c, be1c, wpt, interpret=False):
    D, B = xT.shape
    hid = w0t.shape[0]
    npar = wpt.shape[0]
    grid = (B // _R,)
    full = lambda shape: pl.BlockSpec(shape, lambda i: (0, 0))
    yT, ld = pl.pallas_call(
        _body,
        grid=grid,
        in_specs=[
            pl.BlockSpec((D, _R), lambda i: (0, i)),
            full((D, D)),
            full((hid, D // 2 + 1)), full((hid, 1)), full((hid, 1)),
            full((hid, hid + 1)), full((hid, 1)), full((hid, 1)),
            full((npar, hid + 1)),
        ],
        out_specs=[
            pl.BlockSpec((D, _R), lambda i: (0, i)),
            pl.BlockSpec((1, _R), lambda i: (0, i)),
        ],
        out_shape=[
            jax.ShapeDtypeStruct((D, B), jnp.float32),
            jax.ShapeDtypeStruct((1, B), jnp.float32),
        ],
        compiler_params=pltpu.CompilerParams(
            dimension_semantics=("parallel",)),
        interpret=interpret,
    )(xT, dt, w0t, g0c, be0c, w1t, g1c, be1c, wpt)
    return yT, ld[0]


def kernel(x, W0, b0, g0, be0, W1, b1, g1, be1, Wp, bp):
    B, D = x.shape
    half = D // 2
    hid = W0.shape[1]
    npp = 3 * _N_BINS + 1

    dt_np = np.zeros((D, D), np.float32)
    for u in range(half):
        dt_np[u, 2 * u] = 1.0
        dt_np[half + u, 2 * u + 1] = 1.0
    dt = jnp.asarray(dt_np)

    w0t = jnp.concatenate([W0.T, b0[:, None]], axis=1)
    w1t = jnp.concatenate([W1.T, b1[:, None]], axis=1)
    wpt = Wp.reshape(hid, half, npp).transpose(2, 1, 0).reshape(npp * half, hid)
    bpc = bp.reshape(half, npp).T.reshape(npp * half, 1)
    wpt = jnp.concatenate([wpt, bpc], axis=1)

    yT, ld = _run(x.T, dt, w0t, g0[:, None], be0[:, None],
                  w1t, g1[:, None], be1[:, None], wpt)
    return yT.T, ld
